# baseline (speedup 1.0000x reference)
import functools

import jax
import jax.numpy as jnp
from jax import lax
from jax.experimental import pallas as pl
from jax.experimental.pallas import tpu as pltpu
from jax.experimental.pallas import tpu_sc as plsc

N = 10000
E = 320000
H = 128
OUT = 256
G = 64

NC = 2
NS = 16
NW = NC * NS
CH = 80
NCHUNK = 128
EPWP = CH * NCHUNK
PADE = NW * EPWP
NACC = 10112
NBUF = 4
IDXG = 8
NSTAGE = NCHUNK // IDXG
RPS = NACC // NS

@functools.cache
def _mesh():
  return plsc.VectorSubcoreMesh(
      core_axis_name="c", subcore_axis_name="s", num_cores=NC,
      num_subcores=NS)


@functools.cache
def _make_agg():
  scratch = [
      pltpu.VMEM_SHARED((NACC, H), jnp.float32),
      pltpu.VMEM((IDXG, CH), jnp.int32),
      pltpu.VMEM((IDXG, CH), jnp.int32),
      pltpu.VMEM((NBUF, CH, H), jnp.float32),
      pltpu.SemaphoreType.DMA,
      pltpu.SemaphoreType.DMA,
      pltpu.SemaphoreType.DMA,
  ]

  @functools.partial(
      pl.kernel,
      out_type=jax.ShapeDtypeStruct((NC, NACC, H), jnp.float32),
      mesh=_mesh(),
      scratch_types=scratch,
  )
  def agg(h2_hbm, src_hbm, dst_hbm, zrows_hbm, out_hbm, acc, idx_s, idx_d,
          rows, sem_g, sem_a, sem_b):
    c = lax.axis_index("c")
    s = lax.axis_index("s")
    w = s * NC + c

    pltpu.sync_copy(zrows_hbm, acc.at[pl.ds(s * RPS, RPS)])
    plsc.subcore_barrier()

    def drain(sem, k):
      for _ in range(k):
        pltpu.make_async_copy(zrows_hbm.at[pl.ds(0, CH)], rows.at[0],
                              sem).wait()

    def stage(i, carry):
      pltpu.sync_copy(src_hbm.at[pl.ds(w * NCHUNK + i * IDXG, IDXG)], idx_s)
      pltpu.sync_copy(dst_hbm.at[pl.ds(w * NCHUNK + i * IDXG, IDXG)], idx_d)
      for half in range(2):
        cps = []
        for g, sem_s in ((0, sem_a), (1, sem_b)):

          @pl.when(i + half > 0)
          def _():
            drain(sem_s, 2)

          for b in range(2):
            chunk = half * 4 + g * 2 + b
            cps.append(pltpu.async_copy(h2_hbm.at[c].at[idx_s.at[chunk]],
                                        rows.at[g * 2 + b], sem_g))
        for g, sem_s in ((0, sem_a), (1, sem_b)):
          for b in range(2):
            chunk = half * 4 + g * 2 + b
            cps[g * 2 + b].wait()
            pltpu.async_copy(rows.at[g * 2 + b], acc.at[idx_d.at[chunk]],
                             sem_s, add=True)
      return carry

    lax.fori_loop(0, NSTAGE, stage, 0)
    drain(sem_a, 2)
    drain(sem_b, 2)
    plsc.subcore_barrier()
    pltpu.sync_copy(acc.at[pl.ds(s * RPS, RPS)],
                    out_hbm.at[c, pl.ds(s * RPS, RPS)])

  return agg


@functools.cache
def _make_deg():
  scratch = [
      pltpu.VMEM_SHARED((NACC,), jnp.float32),
      pltpu.VMEM((IDXG, CH), jnp.int32),
      pltpu.VMEM((CH,), jnp.float32),
      pltpu.VMEM((RPS,), jnp.float32),
  ]

  @functools.partial(
      pl.kernel,
      out_type=jax.ShapeDtypeStruct((NC * NACC,), jnp.float32),
      mesh=_mesh(),
      scratch_types=scratch,
  )
  def degk(dst_hbm, zdeg_hbm, ones_hbm, deg_hbm, dacc, idx_d, ones_v, buf):
    c = lax.axis_index("c")
    s = lax.axis_index("s")
    w = s * NC + c

    pltpu.sync_copy(zdeg_hbm, buf)
    pltpu.sync_copy(buf, dacc.at[pl.ds(s * RPS, RPS)])
    pltpu.sync_copy(ones_hbm, ones_v)
    plsc.subcore_barrier()

    def stage(i, carry):
      pltpu.sync_copy(dst_hbm.at[pl.ds(w * NCHUNK + i * IDXG, IDXG)], idx_d)
      for j in range(IDXG):
        pltpu.sync_copy(ones_v, dacc.at[idx_d.at[j]], add=True)
      return carry

    lax.fori_loop(0, NSTAGE, stage, 0)
    plsc.subcore_barrier()
    pltpu.sync_copy(dacc.at[pl.ds(s * RPS, RPS)], buf)
    pltpu.sync_copy(buf, deg_hbm.at[pl.ds(c * NACC + s * RPS, RPS)])

  return degk


BLKR = 1000


def _layer_body(aggp_ref, h_ref, deg_ref, wl_ref, bl_ref, wr_ref, g_ref,
                b_ref, o_ref):
  deg = deg_ref[0] + deg_ref[1] if NC == 2 else deg_ref[0]
  inv = 1.0 / jnp.maximum(deg, 1.0)
  acc = ((aggp_ref[0] + aggp_ref[1]) if NC == 2 else aggp_ref[0]) * inv
  y = lax.dot_general(acc, wl_ref[...], (((1,), (1,)), ((), ())),
                      preferred_element_type=jnp.float32)
  y = y + lax.dot_general(h_ref[...], wr_ref[...], (((1,), (1,)), ((), ())),
                          preferred_element_type=jnp.float32)
  y = y + bl_ref[...]
  mu = jnp.mean(y, axis=-1, keepdims=True)
  yc = y - mu
  var = jnp.mean(yc * yc, axis=-1, keepdims=True)
  o_ref[...] = jnp.maximum(
      yc * lax.rsqrt(var + 1e-5) * g_ref[...] + b_ref[...], 0.0)


def _layer_tc(aggp, h, deg, wl, bl, wr, gg, bb):
  return pl.pallas_call(
      _layer_body,
      grid=(N // BLKR,),
      in_specs=[
          pl.BlockSpec((NC, BLKR, H), lambda i: (0, i, 0)),
          pl.BlockSpec((BLKR, H), lambda i: (i, 0)),
          pl.BlockSpec((NC, BLKR, 1), lambda i: (0, i, 0)),
          pl.BlockSpec((H, H), lambda i: (0, 0)),
          pl.BlockSpec((1, H), lambda i: (0, 0)),
          pl.BlockSpec((H, H), lambda i: (0, 0)),
          pl.BlockSpec((1, H), lambda i: (0, 0)),
          pl.BlockSpec((1, H), lambda i: (0, 0)),
      ],
      out_specs=pl.BlockSpec((BLKR, H), lambda i: (i, 0)),
      out_shape=jax.ShapeDtypeStruct((N, H), jnp.float32),
  )(aggp, h, deg, wl, bl, wr, gg, bb)


PADN = 10240
CHP = 32


def _final_body(h_ref, pb_ref, w1_ref, b1_ref, w2_ref, b2_ref, o_ref,
                pool_ref):
  bm = pb_ref[...]

  def per_g(g, carry):
    sg = jnp.sum(jnp.where(bm < g, 1, 0))
    eg = jnp.sum(jnp.where(bm <= g, 1, 0))
    k0 = sg // CHP
    k1 = (eg + CHP - 1) // CHP

    def inner(k, sm):
      s_acc, m_acc = sm
      rws = h_ref[pl.ds(k * CHP, CHP), :]
      ridx = k * CHP + lax.broadcasted_iota(jnp.int32, (CHP, 1), 0)
      msk = (ridx >= sg) & (ridx < eg)
      s_acc = s_acc + jnp.where(msk, rws, 0.0)
      m_acc = jnp.maximum(m_acc, jnp.where(msk, rws, -jnp.inf))
      return s_acc, m_acc

    s_acc, m_acc = lax.fori_loop(
        k0, k1, inner,
        (jnp.zeros((CHP, H), jnp.float32),
         jnp.full((CHP, H), -jnp.inf, jnp.float32)))
    cnt = (eg - sg).astype(jnp.float32)
    mean = jnp.sum(s_acc, axis=0, keepdims=True) / jnp.maximum(cnt, 1.0)
    mx = jnp.max(m_acc, axis=0, keepdims=True)
    pool_ref[pl.ds(g, 1), :] = jnp.concatenate([mean, mx], axis=1)
    return carry

  lax.fori_loop(0, G, per_g, 0)
  pooled = pool_ref[...]
  r = jnp.maximum(
      lax.dot_general(pooled, w1_ref[...], (((1,), (1,)), ((), ())),
                      preferred_element_type=jnp.float32) + b1_ref[...], 0.0)
  emb = lax.dot_general(r, w2_ref[...], (((1,), (1,)), ((), ())),
                        preferred_element_type=jnp.float32) + b2_ref[...]
  nrm = jnp.sqrt(jnp.sum(emb * emb, axis=-1, keepdims=True))
  o_ref[...] = emb / jnp.maximum(nrm, 1e-12)


def _final_tc(h_pad, pb, w1, b1, w2, b2):
  return pl.pallas_call(
      _final_body,
      out_shape=jax.ShapeDtypeStruct((G, OUT), jnp.float32),
      scratch_shapes=[pltpu.VMEM((G, 2 * H), jnp.float32)],
  )(h_pad, pb, w1, b1, w2, b2)


def kernel(x, edge_index, graph_batch, Wl0, bl0, Wr0, g0, be0, Wl1, bl1, Wr1,
           g1, be1, Wl2, bl2, Wr2, g2, be2, Wro1, bro1, Wro2, bro2):
  src = jnp.concatenate(
      [edge_index[0], jnp.zeros((PADE - E,), jnp.int32)]).reshape(
          NW * NCHUNK, CH)
  dst = jnp.concatenate(
      [edge_index[1], jnp.full((PADE - E,), N, jnp.int32)]).reshape(
          NW * NCHUNK, CH)
  zrows = jnp.zeros((RPS, H), jnp.float32)
  zdeg = jnp.zeros((RPS,), jnp.float32)
  ones = jnp.ones((CH,), jnp.float32)

  deg = _make_deg()(dst, zdeg, ones).reshape(NC, NACC, 1)
  aggp = _make_agg()(jnp.stack([x, x]), src, dst, zrows)
  h = _layer_tc(aggp, x, deg, Wl0, bl0.reshape(1, H), Wr0,
                g0.reshape(1, H), be0.reshape(1, H))
  aggp = _make_agg()(jnp.stack([h, h]), src, dst, zrows)
  h = _layer_tc(aggp, h, deg, Wl1, bl1.reshape(1, H), Wr1,
                g1.reshape(1, H), be1.reshape(1, H))
  aggp = _make_agg()(jnp.stack([h, h]), src, dst, zrows)
  h = _layer_tc(aggp, h, deg, Wl2, bl2.reshape(1, H), Wr2,
                g2.reshape(1, H), be2.reshape(1, H))

  h_pad = jnp.concatenate(
      [h, jnp.zeros((PADN - N, H), jnp.float32)], axis=0)
  pb = jnp.concatenate(
      [graph_batch, jnp.full((PADN - N,), G + 63, jnp.int32)]).reshape(
          PADN // 128, 128)
  return _final_tc(h_pad, pb, Wro1, bro1.reshape(1, H), Wro2,
                   bro2.reshape(1, OUT))


# speedup vs baseline: 1.1861x; 1.1861x over previous
import functools

import jax
import jax.numpy as jnp
from jax import lax
from jax.experimental import pallas as pl
from jax.experimental.pallas import tpu as pltpu
from jax.experimental.pallas import tpu_sc as plsc

N = 10000
E = 320000
H = 128
OUT = 256
G = 64

NC = 2
NS = 16
NW = NC * NS
CH = 80
NCHUNK = 128
EPWP = CH * NCHUNK
PADE = NW * EPWP
NACC = 10112
NBUF = 4
IDXG = 8
NSTAGE = NCHUNK // IDXG
RPS = NACC // NS

@functools.cache
def _mesh():
  return plsc.VectorSubcoreMesh(
      core_axis_name="c", subcore_axis_name="s", num_cores=NC,
      num_subcores=NS)


@functools.cache
def _make_agg():
  scratch = [
      pltpu.VMEM_SHARED((NACC, H), jnp.float32),
      pltpu.VMEM((IDXG, CH), jnp.int32),
      pltpu.VMEM((IDXG, CH), jnp.int32),
      pltpu.VMEM((NBUF, CH, H), jnp.float32),
      pltpu.SemaphoreType.DMA,
      pltpu.SemaphoreType.DMA,
      pltpu.SemaphoreType.DMA,
  ]

  @functools.partial(
      pl.kernel,
      out_type=jax.ShapeDtypeStruct((NC, NACC, H), jnp.float32),
      mesh=_mesh(),
      scratch_types=scratch,
  )
  def agg(h_hbm, src_hbm, dst_hbm, zrows_hbm, out_hbm, acc, idx_s, idx_d,
          rows, sem_g, sem_a, sem_b):
    c = lax.axis_index("c")
    s = lax.axis_index("s")
    w = s * NC + c

    pltpu.sync_copy(zrows_hbm, acc.at[pl.ds(s * RPS, RPS)])
    plsc.subcore_barrier()

    def drain(sem, k):
      for _ in range(k):
        pltpu.make_async_copy(zrows_hbm.at[pl.ds(0, CH)], rows.at[0],
                              sem).wait()

    def stage(i, carry):
      pltpu.sync_copy(src_hbm.at[pl.ds(w * NCHUNK + i * IDXG, IDXG)], idx_s)
      pltpu.sync_copy(dst_hbm.at[pl.ds(w * NCHUNK + i * IDXG, IDXG)], idx_d)
      for half in range(2):
        cps = []
        for g, sem_s in ((0, sem_a), (1, sem_b)):

          @pl.when(i + half > 0)
          def _():
            drain(sem_s, 2)

          for b in range(2):
            chunk = half * 4 + g * 2 + b
            cps.append(pltpu.async_copy(h_hbm.at[idx_s.at[chunk]],
                                        rows.at[g * 2 + b], sem_g))
        for g, sem_s in ((0, sem_a), (1, sem_b)):
          for b in range(2):
            chunk = half * 4 + g * 2 + b
            cps[g * 2 + b].wait()
            pltpu.async_copy(rows.at[g * 2 + b], acc.at[idx_d.at[chunk]],
                             sem_s, add=True)
      return carry

    lax.fori_loop(0, NSTAGE, stage, 0)
    drain(sem_a, 2)
    drain(sem_b, 2)
    plsc.subcore_barrier()
    pltpu.sync_copy(acc.at[pl.ds(s * RPS, RPS)],
                    out_hbm.at[c, pl.ds(s * RPS, RPS)])

  return agg


@functools.cache
def _make_deg():
  scratch = [
      pltpu.VMEM_SHARED((NACC,), jnp.float32),
      pltpu.VMEM((IDXG, CH), jnp.int32),
      pltpu.VMEM((CH,), jnp.float32),
      pltpu.VMEM((RPS,), jnp.float32),
  ]

  @functools.partial(
      pl.kernel,
      out_type=jax.ShapeDtypeStruct((NC * NACC,), jnp.float32),
      mesh=_mesh(),
      scratch_types=scratch,
  )
  def degk(dst_hbm, zdeg_hbm, ones_hbm, deg_hbm, dacc, idx_d, ones_v, buf):
    c = lax.axis_index("c")
    s = lax.axis_index("s")
    w = s * NC + c

    pltpu.sync_copy(zdeg_hbm, buf)
    pltpu.sync_copy(buf, dacc.at[pl.ds(s * RPS, RPS)])
    pltpu.sync_copy(ones_hbm, ones_v)
    plsc.subcore_barrier()

    def stage(i, carry):
      pltpu.sync_copy(dst_hbm.at[pl.ds(w * NCHUNK + i * IDXG, IDXG)], idx_d)
      for j in range(IDXG):
        pltpu.sync_copy(ones_v, dacc.at[idx_d.at[j]], add=True)
      return carry

    lax.fori_loop(0, NSTAGE, stage, 0)
    plsc.subcore_barrier()
    pltpu.sync_copy(dacc.at[pl.ds(s * RPS, RPS)], buf)
    pltpu.sync_copy(buf, deg_hbm.at[pl.ds(c * NACC + s * RPS, RPS)])

  return degk


BLKR = 1000


def _layer_body(aggp_ref, h_ref, deg_ref, wl_ref, bl_ref, wr_ref, g_ref,
                b_ref, o_ref):
  deg = deg_ref[0] + deg_ref[1] if NC == 2 else deg_ref[0]
  inv = 1.0 / jnp.maximum(deg, 1.0)
  acc = ((aggp_ref[0] + aggp_ref[1]) if NC == 2 else aggp_ref[0]) * inv
  y = lax.dot_general(acc, wl_ref[...], (((1,), (1,)), ((), ())),
                      preferred_element_type=jnp.float32)
  y = y + lax.dot_general(h_ref[...], wr_ref[...], (((1,), (1,)), ((), ())),
                          preferred_element_type=jnp.float32)
  y = y + bl_ref[...]
  mu = jnp.mean(y, axis=-1, keepdims=True)
  yc = y - mu
  var = jnp.mean(yc * yc, axis=-1, keepdims=True)
  o_ref[...] = jnp.maximum(
      yc * lax.rsqrt(var + 1e-5) * g_ref[...] + b_ref[...], 0.0)


def _layer_tc(aggp, h, deg, wl, bl, wr, gg, bb):
  return pl.pallas_call(
      _layer_body,
      grid=(N // BLKR,),
      in_specs=[
          pl.BlockSpec((NC, BLKR, H), lambda i: (0, i, 0)),
          pl.BlockSpec((BLKR, H), lambda i: (i, 0)),
          pl.BlockSpec((NC, BLKR, 1), lambda i: (0, i, 0)),
          pl.BlockSpec((H, H), lambda i: (0, 0)),
          pl.BlockSpec((1, H), lambda i: (0, 0)),
          pl.BlockSpec((H, H), lambda i: (0, 0)),
          pl.BlockSpec((1, H), lambda i: (0, 0)),
          pl.BlockSpec((1, H), lambda i: (0, 0)),
      ],
      out_specs=pl.BlockSpec((BLKR, H), lambda i: (i, 0)),
      out_shape=jax.ShapeDtypeStruct((N, H), jnp.float32),
  )(aggp, h, deg, wl, bl, wr, gg, bb)


PADN = 10240
CHP = 32


def _final_body(h_ref, pb_ref, w1_ref, b1_ref, w2_ref, b2_ref, o_ref,
                pool_ref):
  bm = pb_ref[...]

  def per_g(g, carry):
    sg = jnp.sum(jnp.where(bm < g, 1, 0))
    eg = jnp.sum(jnp.where(bm <= g, 1, 0))
    k0 = sg // CHP
    k1 = (eg + CHP - 1) // CHP

    def inner(k, sm):
      s_acc, m_acc = sm
      rws = h_ref[pl.ds(k * CHP, CHP), :]
      ridx = k * CHP + lax.broadcasted_iota(jnp.int32, (CHP, 1), 0)
      msk = (ridx >= sg) & (ridx < eg)
      s_acc = s_acc + jnp.where(msk, rws, 0.0)
      m_acc = jnp.maximum(m_acc, jnp.where(msk, rws, -jnp.inf))
      return s_acc, m_acc

    s_acc, m_acc = lax.fori_loop(
        k0, k1, inner,
        (jnp.zeros((CHP, H), jnp.float32),
         jnp.full((CHP, H), -jnp.inf, jnp.float32)))
    cnt = (eg - sg).astype(jnp.float32)
    mean = jnp.sum(s_acc, axis=0, keepdims=True) / jnp.maximum(cnt, 1.0)
    mx = jnp.max(m_acc, axis=0, keepdims=True)
    pool_ref[pl.ds(g, 1), :] = jnp.concatenate([mean, mx], axis=1)
    return carry

  lax.fori_loop(0, G, per_g, 0)
  pooled = pool_ref[...]
  r = jnp.maximum(
      lax.dot_general(pooled, w1_ref[...], (((1,), (1,)), ((), ())),
                      preferred_element_type=jnp.float32) + b1_ref[...], 0.0)
  emb = lax.dot_general(r, w2_ref[...], (((1,), (1,)), ((), ())),
                        preferred_element_type=jnp.float32) + b2_ref[...]
  nrm = jnp.sqrt(jnp.sum(emb * emb, axis=-1, keepdims=True))
  o_ref[...] = emb / jnp.maximum(nrm, 1e-12)


def _final_tc(h_pad, pb, w1, b1, w2, b2):
  return pl.pallas_call(
      _final_body,
      out_shape=jax.ShapeDtypeStruct((G, OUT), jnp.float32),
      scratch_shapes=[pltpu.VMEM((G, 2 * H), jnp.float32)],
  )(h_pad, pb, w1, b1, w2, b2)


def kernel(x, edge_index, graph_batch, Wl0, bl0, Wr0, g0, be0, Wl1, bl1, Wr1,
           g1, be1, Wl2, bl2, Wr2, g2, be2, Wro1, bro1, Wro2, bro2):
  src = jnp.concatenate(
      [edge_index[0], jnp.zeros((PADE - E,), jnp.int32)]).reshape(
          NW * NCHUNK, CH)
  dst = jnp.concatenate(
      [edge_index[1], jnp.full((PADE - E,), N, jnp.int32)]).reshape(
          NW * NCHUNK, CH)
  zrows = jnp.zeros((RPS, H), jnp.float32)
  zdeg = jnp.zeros((RPS,), jnp.float32)
  ones = jnp.ones((CH,), jnp.float32)

  deg = _make_deg()(dst, zdeg, ones).reshape(NC, NACC, 1)
  aggp = _make_agg()(x, src, dst, zrows)
  h = _layer_tc(aggp, x, deg, Wl0, bl0.reshape(1, H), Wr0,
                g0.reshape(1, H), be0.reshape(1, H))
  aggp = _make_agg()(h, src, dst, zrows)
  h = _layer_tc(aggp, h, deg, Wl1, bl1.reshape(1, H), Wr1,
                g1.reshape(1, H), be1.reshape(1, H))
  aggp = _make_agg()(h, src, dst, zrows)
  h = _layer_tc(aggp, h, deg, Wl2, bl2.reshape(1, H), Wr2,
                g2.reshape(1, H), be2.reshape(1, H))

  h_pad = jnp.concatenate(
      [h, jnp.zeros((PADN - N, H), jnp.float32)], axis=0)
  pb = jnp.concatenate(
      [graph_batch, jnp.full((PADN - N,), G + 63, jnp.int32)]).reshape(
          PADN // 128, 128)
  return _final_tc(h_pad, pb, Wro1, bro1.reshape(1, H), Wro2,
                   bro2.reshape(1, OUT))
